# baseline (speedup 1.0000x reference)
import jax
import jax.numpy as jnp
from jax import lax
from jax.experimental import pallas as pl
from jax.experimental.pallas import tpu as pltpu
from jax.experimental.pallas import tpu_sc as plsc

N_NODES = 100000
N_EDGES = 3200000

_NC = 2
_NS = 16
_NW = _NC * _NS

_ELS = 3 * N_EDGES
_ELS_PER_TILE = _ELS // _NW
_CHUNK_ELS = 30000
_N_CHUNKS = _ELS_PER_TILE // _CHUNK_ELS

_N_ACC3 = 307200
_ACC_PER_TILE = _N_ACC3 // _NS

_BLK_E = 5120
_GRID = N_EDGES // _BLK_E


def _edge_mlp_kernel(x_ref, r_ref, w1bd_ref, b1_ref, w2bd_ref, b2_ref,
                     q_ref, v_ref, s_ref):
    pid = pl.program_id(0)

    @pl.when(pid == 0)
    def _init():
        v_ref[...] = jnp.zeros_like(v_ref)
        s_ref[...] = jnp.zeros_like(s_ref)

    xp = x_ref[...]
    hp = jnp.dot(xp, w1bd_ref[...],
                 preferred_element_type=jnp.float32)
    hp = hp + b1_ref[...]
    hp = hp * (1.0 / (1.0 + jnp.exp(-hp)))
    fmp = jnp.dot(hp, w2bd_ref[...],
                  preferred_element_type=jnp.float32)
    fmp = fmp + b2_ref[0, 0]
    expand = jnp.kron(jnp.eye(8, dtype=jnp.float32),
                      jnp.ones((1, 3), jnp.float32))
    fme = jnp.dot(fmp, expand,
                  preferred_element_type=jnp.float32)
    r8 = r_ref[...]
    q8 = fme * r8
    q_ref[...] = q8
    v_ref[...] += lax.dot_general(q8, r8, (((0,), (0,)), ((), ())),
                                  preferred_element_type=jnp.float32)
    s_ref[...] += jnp.sum(jnp.abs(r8)).reshape(1, 1)


def _edge_mlp(x2d, r8, W1bd, b1rep, W2bd, b2):
    full = lambda i: (0, 0)
    row = lambda i: (i, 0)
    return pl.pallas_call(
        _edge_mlp_kernel,
        grid=(_GRID,),
        in_specs=[
            pl.BlockSpec((_BLK_E // 8, 128), row),
            pl.BlockSpec((_BLK_E // 8, 24), row),
            pl.BlockSpec((128, 256), full),
            pl.BlockSpec((1, 256), full),
            pl.BlockSpec((256, 8), full),
            pl.BlockSpec((1, 1), full),
        ],
        out_specs=[
            pl.BlockSpec((_BLK_E // 8, 24), row),
            pl.BlockSpec((24, 24), full),
            pl.BlockSpec((1, 1), full),
        ],
        out_shape=[
            jax.ShapeDtypeStruct((N_EDGES // 8, 24), jnp.float32),
            jax.ShapeDtypeStruct((24, 24), jnp.float32),
            jax.ShapeDtypeStruct((1, 1), jnp.float32),
        ],
    )(x2d, r8, W1bd, b1rep, W2bd, b2)


def _sc_scatter_body(zeros_hbm, q_hbm, idx_hbm, out_hbm,
                     acc_d, acc_s, gbuf, dbuf, sbuf):
    cid = lax.axis_index("c")
    sid = lax.axis_index("s")
    wid = sid * _NC + cid

    zb = sid * _ACC_PER_TILE
    pltpu.sync_copy(zeros_hbm, acc_d.at[pl.ds(zb, _ACC_PER_TILE)])
    pltpu.sync_copy(zeros_hbm, acc_s.at[pl.ds(zb, _ACC_PER_TILE)])
    plsc.subcore_barrier()

    @pl.loop(0, _N_CHUNKS)
    def _chunk(ci):
        ebase = wid * _ELS_PER_TILE + ci * _CHUNK_ELS
        pltpu.sync_copy(q_hbm.at[pl.ds(ebase, _CHUNK_ELS)], gbuf)
        pltpu.sync_copy(idx_hbm.at[1, pl.ds(ebase, _CHUNK_ELS)], dbuf)
        pltpu.sync_copy(idx_hbm.at[0, pl.ds(ebase, _CHUNK_ELS)], sbuf)
        pltpu.sync_copy(gbuf, acc_d.at[dbuf], add=True)
        pltpu.sync_copy(gbuf, acc_s.at[sbuf], add=True)

    plsc.subcore_barrier()
    pltpu.sync_copy(acc_d.at[pl.ds(zb, _ACC_PER_TILE)],
                    out_hbm.at[cid, 0, pl.ds(zb, _ACC_PER_TILE)])
    pltpu.sync_copy(acc_s.at[pl.ds(zb, _ACC_PER_TILE)],
                    out_hbm.at[cid, 1, pl.ds(zb, _ACC_PER_TILE)])


def _sc_scatter(zeros, qflat, idx3):
    mesh = plsc.VectorSubcoreMesh(core_axis_name="c", subcore_axis_name="s")
    f = pl.kernel(
        _sc_scatter_body,
        out_type=jax.ShapeDtypeStruct((_NC, 2, _N_ACC3), jnp.float32),
        mesh=mesh,
        compiler_params=pltpu.CompilerParams(use_tc_tiling_on_sc=False),
        scratch_types=[
            pltpu.VMEM_SHARED((_N_ACC3,), jnp.float32),
            pltpu.VMEM_SHARED((_N_ACC3,), jnp.float32),
            pltpu.VMEM((_CHUNK_ELS,), jnp.float32),
            pltpu.VMEM((_CHUNK_ELS,), jnp.int32),
            pltpu.VMEM((_CHUNK_ELS,), jnp.int32),
        ],
    )
    return f(zeros, qflat, idx3)


def _combine_kernel(p_ref, v_ref, s_ref, f_ref, vout_ref):
    inv = 1.0 / s_ref[0, 0]
    f = (p_ref[0, 0] + p_ref[1, 0]) - (p_ref[0, 1] + p_ref[1, 1])
    f_ref[...] = f * inv
    v24 = v_ref[...]
    v3 = v24[0:3, 0:3]
    for c in range(1, 8):
        v3 = v3 + v24[3 * c:3 * c + 3, 3 * c:3 * c + 3]
    vout_ref[...] = v3 * inv


def _combine(partials, v24, s):
    p = partials.reshape(_NC, 2, _N_ACC3 // 128, 128)
    return pl.pallas_call(
        _combine_kernel,
        out_shape=[
            jax.ShapeDtypeStruct((_N_ACC3 // 128, 128), jnp.float32),
            jax.ShapeDtypeStruct((3, 3), jnp.float32),
        ],
    )(p, v24, s)


@jax.jit
def kernel(x_ji, r, edge_index, W1, b1, W2, b2):
    eye8 = jnp.eye(8, dtype=jnp.float32)
    q8, v24, s = _edge_mlp(
        x_ji.reshape(-1, 128), r.reshape(-1, 24),
        jnp.kron(eye8, W1), jnp.tile(b1, 8).reshape(1, 256),
        jnp.kron(eye8, W2), b2.reshape(1, 1))

    idx3 = (edge_index[:, :, None] * 3
            + jnp.arange(3, dtype=jnp.int32)).reshape(2, _ELS)

    zeros = jnp.zeros((_ACC_PER_TILE,), jnp.float32)
    partials = _sc_scatter(zeros, q8.reshape(_ELS), idx3)

    f_dense, virial = _combine(partials, v24, s)
    forces = f_dense.reshape(_N_ACC3)[:3 * N_NODES].reshape(N_NODES, 3)
    return forces, virial


# speedup vs baseline: 1.1184x; 1.1184x over previous
import jax
import jax.numpy as jnp
from jax import lax
from jax.experimental import pallas as pl
from jax.experimental.pallas import tpu as pltpu
from jax.experimental.pallas import tpu_sc as plsc

N_NODES = 100000
N_EDGES = 3200000

_NC = 2
_NS = 16
_NW = _NC * _NS

_E_PER_TILE = N_EDGES // _NW
_CHUNK_E = 10000
_N_CHUNKS = _E_PER_TILE // _CHUNK_E

_N_ACC = 102400
_ACC_PER_TILE = _N_ACC // _NS

_BLK_E = 5120
_GRID = N_EDGES // _BLK_E


def _edge_mlp_kernel(x_ref, r_ref, w1bd_ref, b1_ref, w2bd_ref, b2_ref,
                     q_ref, v_ref, s_ref):
    pid = pl.program_id(0)

    @pl.when(pid == 0)
    def _init():
        v_ref[...] = jnp.zeros_like(v_ref)
        s_ref[...] = jnp.zeros_like(s_ref)

    xp = x_ref[...]
    hp = jnp.dot(xp, w1bd_ref[...],
                 preferred_element_type=jnp.float32)
    hp = hp + b1_ref[...]
    hp = hp * (1.0 / (1.0 + jnp.exp(-hp)))
    fmp = jnp.dot(hp, w2bd_ref[...],
                  preferred_element_type=jnp.float32)
    fmp = fmp + b2_ref[0, 0]
    eye8 = jnp.eye(8, dtype=jnp.float32)
    expand = jnp.kron(eye8, jnp.ones((1, 3), jnp.float32))
    fme = jnp.dot(fmp, expand,
                  preferred_element_type=jnp.float32)
    r8 = r_ref[...]
    q8 = fme * r8
    eye3 = jnp.eye(3, dtype=jnp.float32)
    for c in range(3):
        q_ref[c, :, :] = jnp.dot(
            q8, jnp.kron(eye8, eye3[:, c:c + 1]),
            preferred_element_type=jnp.float32)
    v_ref[...] += lax.dot_general(q8, r8, (((0,), (0,)), ((), ())),
                                  preferred_element_type=jnp.float32)
    s_ref[...] += jnp.sum(jnp.abs(r8)).reshape(1, 1)


def _edge_mlp(x2d, r8, W1bd, b1rep, W2bd, b2):
    full = lambda i: (0, 0)
    row = lambda i: (i, 0)
    return pl.pallas_call(
        _edge_mlp_kernel,
        grid=(_GRID,),
        in_specs=[
            pl.BlockSpec((_BLK_E // 8, 128), row),
            pl.BlockSpec((_BLK_E // 8, 24), row),
            pl.BlockSpec((128, 256), full),
            pl.BlockSpec((1, 256), full),
            pl.BlockSpec((256, 8), full),
            pl.BlockSpec((1, 1), full),
        ],
        out_specs=[
            pl.BlockSpec((3, _BLK_E // 8, 8), lambda i: (0, i, 0)),
            pl.BlockSpec((24, 24), full),
            pl.BlockSpec((1, 1), full),
        ],
        out_shape=[
            jax.ShapeDtypeStruct((3, N_EDGES // 8, 8), jnp.float32),
            jax.ShapeDtypeStruct((24, 24), jnp.float32),
            jax.ShapeDtypeStruct((1, 1), jnp.float32),
        ],
    )(x2d, r8, W1bd, b1rep, W2bd, b2)


def _sc_scatter_body(zeros_hbm, q_hbm, idx_hbm, out_hbm,
                     adx, ady, adz, asx, asy, asz, gbuf, dbuf, sbuf):
    cid = lax.axis_index("c")
    sid = lax.axis_index("s")
    wid = sid * _NC + cid

    zb = sid * _ACC_PER_TILE
    for acc in (adx, ady, adz, asx, asy, asz):
        pltpu.sync_copy(zeros_hbm, acc.at[pl.ds(zb, _ACC_PER_TILE)])
    plsc.subcore_barrier()

    @pl.loop(0, _N_CHUNKS)
    def _chunk(ci):
        ebase = wid * _E_PER_TILE + ci * _CHUNK_E
        pltpu.sync_copy(q_hbm.at[:, pl.ds(ebase, _CHUNK_E)], gbuf)
        pltpu.sync_copy(idx_hbm.at[1, pl.ds(ebase, _CHUNK_E)], dbuf)
        pltpu.sync_copy(idx_hbm.at[0, pl.ds(ebase, _CHUNK_E)], sbuf)
        for c, (ad, as_) in enumerate(((adx, asx), (ady, asy), (adz, asz))):
            pltpu.sync_copy(gbuf.at[c], ad.at[dbuf], add=True)
            pltpu.sync_copy(gbuf.at[c], as_.at[sbuf], add=True)

    plsc.subcore_barrier()
    for p, acc in enumerate((adx, ady, adz, asx, asy, asz)):
        pltpu.sync_copy(acc.at[pl.ds(zb, _ACC_PER_TILE)],
                        out_hbm.at[cid, p, pl.ds(zb, _ACC_PER_TILE)])


def _sc_scatter(zeros, qpl, idx):
    mesh = plsc.VectorSubcoreMesh(core_axis_name="c", subcore_axis_name="s")
    f = pl.kernel(
        _sc_scatter_body,
        out_type=jax.ShapeDtypeStruct((_NC, 6, _N_ACC), jnp.float32),
        mesh=mesh,
        compiler_params=pltpu.CompilerParams(use_tc_tiling_on_sc=False),
        scratch_types=[
            pltpu.VMEM_SHARED((_N_ACC,), jnp.float32),
            pltpu.VMEM_SHARED((_N_ACC,), jnp.float32),
            pltpu.VMEM_SHARED((_N_ACC,), jnp.float32),
            pltpu.VMEM_SHARED((_N_ACC,), jnp.float32),
            pltpu.VMEM_SHARED((_N_ACC,), jnp.float32),
            pltpu.VMEM_SHARED((_N_ACC,), jnp.float32),
            pltpu.VMEM((3, _CHUNK_E), jnp.float32),
            pltpu.VMEM((_CHUNK_E,), jnp.int32),
            pltpu.VMEM((_CHUNK_E,), jnp.int32),
        ],
    )
    return f(zeros, qpl, idx)


def _combine_kernel(p_ref, v_ref, s_ref, f_ref, vout_ref):
    inv = 1.0 / s_ref[0, 0]
    planes = [(p_ref[0, c] + p_ref[1, c]) - (p_ref[0, c + 3] + p_ref[1, c + 3])
              for c in range(3)]
    f_ref[...] = jnp.stack(planes) * inv
    v24 = v_ref[...]
    v3 = v24[0:3, 0:3]
    for c in range(1, 8):
        v3 = v3 + v24[3 * c:3 * c + 3, 3 * c:3 * c + 3]
    vout_ref[...] = v3 * inv


def _combine(partials, v24, s):
    return pl.pallas_call(
        _combine_kernel,
        out_shape=[
            jax.ShapeDtypeStruct((3, _N_ACC), jnp.float32),
            jax.ShapeDtypeStruct((3, 3), jnp.float32),
        ],
    )(partials, v24, s)


@jax.jit
def kernel(x_ji, r, edge_index, W1, b1, W2, b2):
    eye8 = jnp.eye(8, dtype=jnp.float32)
    qpl, v24, s = _edge_mlp(
        x_ji.reshape(-1, 128), r.reshape(-1, 24),
        jnp.kron(eye8, W1), jnp.tile(b1, 8).reshape(1, 256),
        jnp.kron(eye8, W2), b2.reshape(1, 1))

    zeros = jnp.zeros((_ACC_PER_TILE,), jnp.float32)
    partials = _sc_scatter(zeros, qpl.reshape(3, N_EDGES), edge_index)

    f_planar, virial = _combine(partials, v24, s)
    forces = f_planar[:, :N_NODES].T
    return forces, virial
